# baseline (speedup 1.0000x reference)
import jax
import jax.numpy as jnp
from jax.experimental import pallas as pl

BATCH = 128
IN_F = 4096
R = 64
OUT_F = 4096
E = 64

EB = 8
JB = 512


def _stage1(wids_ref, x_ref, a_ref, u_ref):
    g = pl.program_id(0)
    x = x_ref[...]
    wids = wids_ref[...]
    for e in range(EB):
        u = jnp.dot(x, a_ref[e], preferred_element_type=jnp.float32)
        sel = wids == (g * EB + e)
        u = jnp.where(sel, u, 0.0)
        u_ref[:, e * R:(e + 1) * R] = u.astype(jnp.bfloat16)


def _stage2(x_ref, u_ref, m_ref, b_ref, o_ref):
    acc = jnp.dot(x_ref[...], m_ref[...], preferred_element_type=jnp.float32)
    acc += 2.0 * jnp.dot(u_ref[...], b_ref[...],
                         preferred_element_type=jnp.float32)
    o_ref[...] = acc.astype(jnp.float32)


def kernel(x, wids, lora_A, lora_B, M):
    x2 = x.reshape(BATCH, IN_F).astype(jnp.bfloat16)
    wids2 = wids.reshape(BATCH, 1)
    a_bf = lora_A.astype(jnp.bfloat16)
    b_flat = lora_B.reshape(E * R, OUT_F).astype(jnp.bfloat16)
    m_bf = M.astype(jnp.bfloat16)

    u = pl.pallas_call(
        _stage1,
        grid=(E // EB,),
        in_specs=[
            pl.BlockSpec((BATCH, 1), lambda g: (0, 0)),
            pl.BlockSpec((BATCH, IN_F), lambda g: (0, 0)),
            pl.BlockSpec((EB, IN_F, R), lambda g: (g, 0, 0)),
        ],
        out_specs=pl.BlockSpec((BATCH, EB * R), lambda g: (0, g)),
        out_shape=jax.ShapeDtypeStruct((BATCH, E * R), jnp.bfloat16),
    )(wids2, x2, a_bf)

    y = pl.pallas_call(
        _stage2,
        grid=(OUT_F // JB,),
        in_specs=[
            pl.BlockSpec((BATCH, IN_F), lambda j: (0, 0)),
            pl.BlockSpec((BATCH, E * R), lambda j: (0, 0)),
            pl.BlockSpec((IN_F, JB), lambda j: (0, j)),
            pl.BlockSpec((E * R, JB), lambda j: (0, j)),
        ],
        out_specs=pl.BlockSpec((BATCH, JB), lambda j: (0, j)),
        out_shape=jax.ShapeDtypeStruct((BATCH, OUT_F), jnp.float32),
    )(x2, u, m_bf, b_flat)

    return y.reshape(BATCH, 1, OUT_F).astype(jnp.float16)
